# baseline (speedup 1.0000x reference)
import functools

import jax
import jax.numpy as jnp
from jax import lax
from jax.experimental import pallas as pl
from jax.experimental.pallas import tpu as pltpu
from jax.experimental.pallas import tpu_sc as plsc

NC = 2
NS = 16
CHUNK = 125
GSZ = 1


def _rnd(a):
    return a.astype(jnp.bfloat16).astype(jnp.float32)


def _dot(a, b):
    return jnp.dot(a, b, preferred_element_type=jnp.float32,
                   precision=lax.Precision.HIGHEST)


def _segment_sum_sc(hb, src4, dst4, zeros):
    N, D = hb.shape
    ngroup, gsz, chunk = src4.shape[1], src4.shape[2], src4.shape[3]
    stripe = (N // NS) & ~7
    rem = N - stripe * NS

    mesh = plsc.VectorSubcoreMesh(core_axis_name="c", subcore_axis_name="s")

    @functools.partial(
        pl.kernel,
        out_type=jax.ShapeDtypeStruct((NC, N, D), jnp.float32),
        mesh=mesh,
        scratch_types=[
            pltpu.VMEM((gsz, chunk), jnp.int32),
            pltpu.VMEM((gsz, chunk), jnp.int32),
            pltpu.VMEM((gsz, chunk), jnp.int32),
            pltpu.VMEM((gsz, chunk), jnp.int32),
            pltpu.VMEM((gsz, chunk, D), jnp.float32),
            pltpu.VMEM((gsz, chunk, D), jnp.float32),
            pltpu.VMEM_SHARED((N, D), jnp.float32),
            pltpu.SemaphoreType.DMA((gsz,)),
            pltpu.SemaphoreType.DMA((gsz,)),
            pltpu.SemaphoreType.DMA((gsz,)),
            pltpu.SemaphoreType.DMA((gsz,)),
            pltpu.SemaphoreType.DMA((2,)),
            pltpu.SemaphoreType.DMA((2,)),
        ],
    )
    def segsum(h_hbm, src_hbm, dst_hbm, z_hbm, out_hbm,
               src0, src1, dst0, dst1, rows0, rows1, acc_sh,
               gsem0, gsem1, ssem0, ssem1, isem0, isem1):
        cid = lax.axis_index("c")
        sid = lax.axis_index("s")
        wid = sid * NC + cid
        pltpu.sync_copy(z_hbm.at[pl.ds(sid * stripe, stripe)],
                        acc_sh.at[pl.ds(sid * stripe, stripe)])
        @pl.when(sid == 0)
        def _():
            pltpu.sync_copy(z_hbm.at[pl.ds(stripe * NS, rem)],
                            acc_sh.at[pl.ds(stripe * NS, rem)])
        plsc.subcore_barrier()

        def gather(src_v, rows_v, gsem, b):
            pltpu.async_copy(h_hbm.at[src_v.at[b]], rows_v.at[b], gsem.at[b])

        def gwait(rows_v, gsem, b):
            pltpu.make_async_copy(h_hbm.at[src0.at[0]], rows_v.at[b],
                                  gsem.at[b]).wait()

        def scat(rows_v, dst_v, ssem, b):
            pltpu.async_copy(rows_v.at[b], acc_sh.at[dst_v.at[b]],
                             ssem.at[b], add=True)

        def swait(rows_v, dst_v, ssem, b):
            pltpu.make_async_copy(rows_v.at[b], acc_sh.at[dst_v.at[0]],
                                  ssem.at[b]).wait()

        def iload(g, v, hbm, isem, half):
            pltpu.async_copy(hbm.at[wid, g], v, isem.at[half])

        def iwait(v, hbm, isem, half):
            pltpu.make_async_copy(hbm.at[0, 0], v, isem.at[half]).wait()

        pltpu.sync_copy(src_hbm.at[wid, 0], src0)
        iload(0, dst0, dst_hbm, isem0, 1)
        iload(1, src1, src_hbm, isem1, 0)
        iload(1, dst1, dst_hbm, isem1, 1)
        for b in range(gsz):
            gather(src0, rows0, gsem0, b)

        def body(k, carry):
            a = 2 * k
            iwait(src1, src_hbm, isem1, 0)
            for b in range(gsz):
                gather(src1, rows1, gsem1, b)
            for b in range(gsz):
                gwait(rows0, gsem0, b)
            @pl.when(a + 2 < ngroup)
            def _():
                iload(a + 2, src0, src_hbm, isem0, 0)
            iwait(dst0, dst_hbm, isem0, 1)
            for b in range(gsz):
                scat(rows0, dst0, ssem0, b)
            for b in range(gsz):
                gwait(rows1, gsem1, b)
            @pl.when(a + 3 < ngroup)
            def _():
                iload(a + 3, src1, src_hbm, isem1, 0)
            iwait(dst1, dst_hbm, isem1, 1)
            for b in range(gsz):
                scat(rows1, dst1, ssem1, b)
            for b in range(gsz):
                swait(rows0, dst0, ssem0, b)
            @pl.when(a + 2 < ngroup)
            def _():
                iload(a + 2, dst0, dst_hbm, isem0, 1)
                iwait(src0, src_hbm, isem0, 0)
                for b in range(gsz):
                    gather(src0, rows0, gsem0, b)
            for b in range(gsz):
                swait(rows1, dst1, ssem1, b)
            @pl.when(a + 3 < ngroup)
            def _():
                iload(a + 3, dst1, dst_hbm, isem1, 1)
            return carry

        lax.fori_loop(0, ngroup // 2, body, 0)
        plsc.subcore_barrier()
        pltpu.sync_copy(acc_sh.at[pl.ds(sid * stripe, stripe)],
                        out_hbm.at[cid, pl.ds(sid * stripe, stripe)])
        @pl.when(sid == 0)
        def _():
            pltpu.sync_copy(acc_sh.at[pl.ds(stripe * NS, rem)],
                            out_hbm.at[cid, pl.ds(stripe * NS, rem)])

    return segsum(hb, src4, dst4, zeros)


def _encoder(x, W_enc, b_enc, blk):
    N, D = x.shape

    def body(x_ref, w_ref, b_ref, o_ref, ob_ref):
        h = jnp.maximum(_dot(_rnd(x_ref[...]), w_ref[...]) + b_ref[...], 0.0)
        o_ref[...] = h
        ob_ref[...] = _rnd(h)

    return pl.pallas_call(
        body,
        grid=(N // blk,),
        in_specs=[pl.BlockSpec((blk, D), lambda i: (i, 0)),
                  pl.BlockSpec((D, D), lambda i: (0, 0)),
                  pl.BlockSpec((1, D), lambda i: (0, 0))],
        out_specs=[pl.BlockSpec((blk, D), lambda i: (i, 0)),
                   pl.BlockSpec((blk, D), lambda i: (i, 0))],
        out_shape=[jax.ShapeDtypeStruct((N, D), jnp.float32),
                   jax.ShapeDtypeStruct((N, D), jnp.float32)],
    )(x, _rnd(W_enc), b_enc.reshape(1, D))


def _node_update(P, We, Wn, h, blk):
    N, D = h.shape

    def body(p_ref, we_ref, wn_ref, h_ref, o_ref, ob_ref):
        s = p_ref[0] + p_ref[1]
        agg = _dot(s, we_ref[...])
        hn = jnp.maximum(_dot(_rnd(agg), wn_ref[...]) + h_ref[...], 0.0)
        o_ref[...] = hn
        ob_ref[...] = _rnd(hn)

    return pl.pallas_call(
        body,
        grid=(N // blk,),
        in_specs=[pl.BlockSpec((2, blk, D), lambda i: (0, i, 0)),
                  pl.BlockSpec((D, D), lambda i: (0, 0)),
                  pl.BlockSpec((D, D), lambda i: (0, 0)),
                  pl.BlockSpec((blk, D), lambda i: (i, 0))],
        out_specs=[pl.BlockSpec((blk, D), lambda i: (i, 0)),
                   pl.BlockSpec((blk, D), lambda i: (i, 0))],
        out_shape=[jax.ShapeDtypeStruct((N, D), jnp.float32),
                   jax.ShapeDtypeStruct((N, D), jnp.float32)],
    )(P, _rnd(We), _rnd(Wn), h)


def _node_update_decoder(P, We, Wn, h, W1, b1, W2, b2, W3, b3, blk):
    N, D = h.shape
    D2, D4 = W1.shape[1], W2.shape[1]

    def body(p_ref, we_ref, wn_ref, h_ref, w1_ref, b1_ref, w2_ref, b2_ref,
             w3_ref, b3_ref, o_ref):
        s = p_ref[0] + p_ref[1]
        agg = _dot(s, we_ref[...])
        hn = jnp.maximum(_dot(_rnd(agg), wn_ref[...]) + h_ref[...], 0.0)
        t = jnp.maximum(_dot(_rnd(hn), w1_ref[...]) + b1_ref[...], 0.0)
        t = jnp.maximum(_dot(_rnd(t), w2_ref[...]) + b2_ref[...], 0.0)
        o_ref[...] = _dot(_rnd(t), w3_ref[...]) + b3_ref[...]

    return pl.pallas_call(
        body,
        grid=(N // blk,),
        in_specs=[pl.BlockSpec((2, blk, D), lambda i: (0, i, 0)),
                  pl.BlockSpec((D, D), lambda i: (0, 0)),
                  pl.BlockSpec((D, D), lambda i: (0, 0)),
                  pl.BlockSpec((blk, D), lambda i: (i, 0)),
                  pl.BlockSpec((D, D2), lambda i: (0, 0)),
                  pl.BlockSpec((1, D2), lambda i: (0, 0)),
                  pl.BlockSpec((D2, D4), lambda i: (0, 0)),
                  pl.BlockSpec((1, D4), lambda i: (0, 0)),
                  pl.BlockSpec((D4, 1), lambda i: (0, 0)),
                  pl.BlockSpec((1, 1), lambda i: (0, 0))],
        out_specs=pl.BlockSpec((blk, 1), lambda i: (i, 0)),
        out_shape=jax.ShapeDtypeStruct((N, 1), jnp.float32),
    )(P, _rnd(We), _rnd(Wn), h, _rnd(W1), b1.reshape(1, D2), _rnd(W2),
      b2.reshape(1, D4), _rnd(W3), b3.reshape(1, 1))


def kernel(x, edge_index, W_enc, b_enc, W_edge, W_node, W1, b1, W2, b2, W3, b3):
    N, D = x.shape
    E = edge_index.shape[1]
    n_layers = W_edge.shape[0]
    blk = 1000

    ngroup = E // (CHUNK * GSZ * NC * NS)
    src4 = edge_index[0].reshape(NC * NS, ngroup, GSZ, CHUNK)
    dst4 = edge_index[1].reshape(NC * NS, ngroup, GSZ, CHUNK)
    zeros = jnp.zeros((N, D), jnp.float32)

    h, hb = _encoder(x, W_enc, b_enc, blk)
    for l in range(n_layers - 1):
        P = _segment_sum_sc(hb, src4, dst4, zeros)
        h, hb = _node_update(P, W_edge[l], W_node[l], h, blk)
    P = _segment_sum_sc(hb, src4, dst4, zeros)
    return _node_update_decoder(P, W_edge[n_layers - 1], W_node[n_layers - 1],
                                h, W1, b1, W2, b2, W3, b3, blk)


# speedup vs baseline: 1.0608x; 1.0608x over previous
import functools

import jax
import jax.numpy as jnp
from jax import lax
from jax.experimental import pallas as pl
from jax.experimental.pallas import tpu as pltpu
from jax.experimental.pallas import tpu_sc as plsc

NC = 2
NS = 16
CHUNK = 125
GSZ = 1


def _rnd(a):
    return a.astype(jnp.bfloat16).astype(jnp.float32)


def _dot(a, b):
    return jnp.dot(a, b, preferred_element_type=jnp.float32,
                   precision=lax.Precision.HIGHEST)


def _segment_sum_sc(hb, src4, dst4, zeros):
    N, D = hb.shape
    ngroup, gsz, chunk = src4.shape[1], src4.shape[2], src4.shape[3]
    stripe = (N // NS) & ~7
    rem = N - stripe * NS

    mesh = plsc.VectorSubcoreMesh(core_axis_name="c", subcore_axis_name="s")

    @functools.partial(
        pl.kernel,
        out_type=jax.ShapeDtypeStruct((NC, N, D), jnp.float32),
        mesh=mesh,
        scratch_types=[
            pltpu.VMEM((gsz, chunk), jnp.int32),
            pltpu.VMEM((gsz, chunk), jnp.int32),
            pltpu.VMEM((gsz, chunk), jnp.int32),
            pltpu.VMEM((gsz, chunk), jnp.int32),
            pltpu.VMEM((gsz, chunk, D), jnp.float32),
            pltpu.VMEM((gsz, chunk, D), jnp.float32),
            pltpu.VMEM_SHARED((N, D), jnp.float32),
            pltpu.SemaphoreType.DMA((gsz,)),
            pltpu.SemaphoreType.DMA((gsz,)),
            pltpu.SemaphoreType.DMA((gsz,)),
            pltpu.SemaphoreType.DMA((gsz,)),
            pltpu.SemaphoreType.DMA((2,)),
            pltpu.SemaphoreType.DMA((2,)),
        ],
    )
    def segsum(h_hbm, src_hbm, dst_hbm, z_hbm, out_hbm,
               src0, src1, dst0, dst1, rows0, rows1, acc_sh,
               gsem0, gsem1, ssem0, ssem1, isem0, isem1):
        cid = lax.axis_index("c")
        sid = lax.axis_index("s")
        wid = sid * NC + cid
        pltpu.sync_copy(z_hbm.at[pl.ds(sid * stripe, stripe)],
                        acc_sh.at[pl.ds(sid * stripe, stripe)])
        @pl.when(sid == 0)
        def _():
            pltpu.sync_copy(z_hbm.at[pl.ds(stripe * NS, rem)],
                            acc_sh.at[pl.ds(stripe * NS, rem)])
        plsc.subcore_barrier()

        def gather(src_v, rows_v, gsem, b):
            pltpu.async_copy(h_hbm.at[src_v.at[b]], rows_v.at[b], gsem.at[b])

        def gwait(rows_v, gsem, b):
            pltpu.make_async_copy(h_hbm.at[src0.at[0]], rows_v.at[b],
                                  gsem.at[b]).wait()

        def scat(rows_v, dst_v, ssem, b):
            pltpu.async_copy(rows_v.at[b], acc_sh.at[dst_v.at[b]],
                             ssem.at[b], add=True)

        def swait(rows_v, dst_v, ssem, b):
            pltpu.make_async_copy(rows_v.at[b], acc_sh.at[dst_v.at[0]],
                                  ssem.at[b]).wait()

        def iload(g, v, hbm, isem, half):
            pltpu.async_copy(hbm.at[wid, g], v, isem.at[half])

        def iwait(v, hbm, isem, half):
            pltpu.make_async_copy(hbm.at[0, 0], v, isem.at[half]).wait()

        pltpu.sync_copy(src_hbm.at[wid, 0], src0)
        pltpu.sync_copy(dst_hbm.at[wid, 0], dst0)
        iload(1, src1, src_hbm, isem1, 0)
        iload(1, dst1, dst_hbm, isem1, 1)
        for b in range(gsz):
            gather(src0, rows0, gsem0, b)

        def body(k, carry):
            a = 2 * k
            for b in range(gsz):
                gwait(rows0, gsem0, b)
            @pl.when(a + 2 < ngroup)
            def _():
                iload(a + 2, src0, src_hbm, isem0, 0)
            for b in range(gsz):
                scat(rows0, dst0, ssem0, b)
            iwait(src1, src_hbm, isem1, 0)
            iwait(dst1, dst_hbm, isem1, 1)
            for b in range(gsz):
                gather(src1, rows1, gsem1, b)
            for b in range(gsz):
                swait(rows0, dst0, ssem0, b)
            @pl.when(a + 2 < ngroup)
            def _():
                iload(a + 2, dst0, dst_hbm, isem0, 1)
            for b in range(gsz):
                gwait(rows1, gsem1, b)
            @pl.when(a + 3 < ngroup)
            def _():
                iload(a + 3, src1, src_hbm, isem1, 0)
            for b in range(gsz):
                scat(rows1, dst1, ssem1, b)
            @pl.when(a + 2 < ngroup)
            def _():
                iwait(src0, src_hbm, isem0, 0)
                iwait(dst0, dst_hbm, isem0, 1)
                for b in range(gsz):
                    gather(src0, rows0, gsem0, b)
            for b in range(gsz):
                swait(rows1, dst1, ssem1, b)
            @pl.when(a + 3 < ngroup)
            def _():
                iload(a + 3, dst1, dst_hbm, isem1, 1)
            return carry

        lax.fori_loop(0, ngroup // 2, body, 0)
        plsc.subcore_barrier()
        pltpu.sync_copy(acc_sh.at[pl.ds(sid * stripe, stripe)],
                        out_hbm.at[cid, pl.ds(sid * stripe, stripe)])
        @pl.when(sid == 0)
        def _():
            pltpu.sync_copy(acc_sh.at[pl.ds(stripe * NS, rem)],
                            out_hbm.at[cid, pl.ds(stripe * NS, rem)])

    return segsum(hb, src4, dst4, zeros)


def _encoder(x, W_enc, b_enc, blk):
    N, D = x.shape

    def body(x_ref, w_ref, b_ref, o_ref, ob_ref):
        h = jnp.maximum(_dot(_rnd(x_ref[...]), w_ref[...]) + b_ref[...], 0.0)
        o_ref[...] = h
        ob_ref[...] = _rnd(h)

    return pl.pallas_call(
        body,
        grid=(N // blk,),
        in_specs=[pl.BlockSpec((blk, D), lambda i: (i, 0)),
                  pl.BlockSpec((D, D), lambda i: (0, 0)),
                  pl.BlockSpec((1, D), lambda i: (0, 0))],
        out_specs=[pl.BlockSpec((blk, D), lambda i: (i, 0)),
                   pl.BlockSpec((blk, D), lambda i: (i, 0))],
        out_shape=[jax.ShapeDtypeStruct((N, D), jnp.float32),
                   jax.ShapeDtypeStruct((N, D), jnp.float32)],
    )(x, _rnd(W_enc), b_enc.reshape(1, D))


def _node_update(P, We, Wn, h, blk):
    N, D = h.shape

    def body(p_ref, we_ref, wn_ref, h_ref, o_ref, ob_ref):
        s = p_ref[0] + p_ref[1]
        agg = _dot(s, we_ref[...])
        hn = jnp.maximum(_dot(_rnd(agg), wn_ref[...]) + h_ref[...], 0.0)
        o_ref[...] = hn
        ob_ref[...] = _rnd(hn)

    return pl.pallas_call(
        body,
        grid=(N // blk,),
        in_specs=[pl.BlockSpec((2, blk, D), lambda i: (0, i, 0)),
                  pl.BlockSpec((D, D), lambda i: (0, 0)),
                  pl.BlockSpec((D, D), lambda i: (0, 0)),
                  pl.BlockSpec((blk, D), lambda i: (i, 0))],
        out_specs=[pl.BlockSpec((blk, D), lambda i: (i, 0)),
                   pl.BlockSpec((blk, D), lambda i: (i, 0))],
        out_shape=[jax.ShapeDtypeStruct((N, D), jnp.float32),
                   jax.ShapeDtypeStruct((N, D), jnp.float32)],
    )(P, _rnd(We), _rnd(Wn), h)


def _node_update_decoder(P, We, Wn, h, W1, b1, W2, b2, W3, b3, blk):
    N, D = h.shape
    D2, D4 = W1.shape[1], W2.shape[1]

    def body(p_ref, we_ref, wn_ref, h_ref, w1_ref, b1_ref, w2_ref, b2_ref,
             w3_ref, b3_ref, o_ref):
        s = p_ref[0] + p_ref[1]
        agg = _dot(s, we_ref[...])
        hn = jnp.maximum(_dot(_rnd(agg), wn_ref[...]) + h_ref[...], 0.0)
        t = jnp.maximum(_dot(_rnd(hn), w1_ref[...]) + b1_ref[...], 0.0)
        t = jnp.maximum(_dot(_rnd(t), w2_ref[...]) + b2_ref[...], 0.0)
        o_ref[...] = _dot(_rnd(t), w3_ref[...]) + b3_ref[...]

    return pl.pallas_call(
        body,
        grid=(N // blk,),
        in_specs=[pl.BlockSpec((2, blk, D), lambda i: (0, i, 0)),
                  pl.BlockSpec((D, D), lambda i: (0, 0)),
                  pl.BlockSpec((D, D), lambda i: (0, 0)),
                  pl.BlockSpec((blk, D), lambda i: (i, 0)),
                  pl.BlockSpec((D, D2), lambda i: (0, 0)),
                  pl.BlockSpec((1, D2), lambda i: (0, 0)),
                  pl.BlockSpec((D2, D4), lambda i: (0, 0)),
                  pl.BlockSpec((1, D4), lambda i: (0, 0)),
                  pl.BlockSpec((D4, 1), lambda i: (0, 0)),
                  pl.BlockSpec((1, 1), lambda i: (0, 0))],
        out_specs=pl.BlockSpec((blk, 1), lambda i: (i, 0)),
        out_shape=jax.ShapeDtypeStruct((N, 1), jnp.float32),
    )(P, _rnd(We), _rnd(Wn), h, _rnd(W1), b1.reshape(1, D2), _rnd(W2),
      b2.reshape(1, D4), _rnd(W3), b3.reshape(1, 1))


def kernel(x, edge_index, W_enc, b_enc, W_edge, W_node, W1, b1, W2, b2, W3, b3):
    N, D = x.shape
    E = edge_index.shape[1]
    n_layers = W_edge.shape[0]
    blk = 1000

    ngroup = E // (CHUNK * GSZ * NC * NS)
    src4 = edge_index[0].reshape(NC * NS, ngroup, GSZ, CHUNK)
    dst4 = edge_index[1].reshape(NC * NS, ngroup, GSZ, CHUNK)
    zeros = jnp.zeros((N, D), jnp.float32)

    h, hb = _encoder(x, W_enc, b_enc, blk)
    for l in range(n_layers - 1):
        P = _segment_sum_sc(hb, src4, dst4, zeros)
        h, hb = _node_update(P, W_edge[l], W_node[l], h, blk)
    P = _segment_sum_sc(hb, src4, dst4, zeros)
    return _node_update_decoder(P, W_edge[n_layers - 1], W_node[n_layers - 1],
                                h, W1, b1, W2, b2, W3, b3, blk)
